# baseline (speedup 1.0000x reference)
import functools

import jax
import jax.numpy as jnp
from jax import lax
from jax.experimental import pallas as pl
from jax.experimental.pallas import tpu as pltpu
from jax.experimental.pallas import tpu_sc as plsc

L_SEQ = 16384
WDIM = 64
PDIM = 64
ODIM = WDIM + PDIM


NUM_CORES = 2
NUM_SUBCORES = 16
NW = NUM_CORES * NUM_SUBCORES
B_PER_W = 288
SC_ROWS = NW * B_PER_W
CH = 96
NQ = 4

_mesh = plsc.VectorSubcoreMesh(core_axis_name="c", subcore_axis_name="s")


@functools.partial(
    pl.kernel,
    mesh=_mesh,
    out_type=jax.ShapeDtypeStruct((SC_ROWS, ODIM), jnp.float32),
    compiler_params=pltpu.CompilerParams(needs_layout_passes=False),
    scratch_types=[
        pltpu.VMEM((B_PER_W,), jnp.int32),
        pltpu.VMEM((CH, WDIM), jnp.float32),
        pltpu.VMEM((CH, WDIM), jnp.float32),
        pltpu.VMEM((CH, PDIM), jnp.float32),
        pltpu.VMEM((CH, PDIM), jnp.float32),
        pltpu.VMEM((CH, ODIM), jnp.float32),
        pltpu.VMEM((CH, ODIM), jnp.float32),
        [pltpu.SemaphoreType.DMA] * (2 * NQ),
        pltpu.SemaphoreType.DMA,
        pltpu.SemaphoreType.DMA,
        pltpu.SemaphoreType.DMA,
    ],
)
def _embed_sc(idx_hbm, wword_hbm, wpos_hbm, out_hbm,
              idx_v, rows0_v, rows1_v, pos0_v, pos1_v, buf0_v, buf1_v,
              gsems, psem0, psem1, wsem):
    rows_b = (rows0_v, rows1_v)
    pos_b = (pos0_v, pos1_v)
    buf_b = (buf0_v, buf1_v)
    psem_b = (psem0, psem1)
    wid = lax.axis_index("s") * NUM_CORES + lax.axis_index("c")
    base = wid * B_PER_W
    pltpu.sync_copy(idx_hbm.at[pl.ds(base, B_PER_W)], idx_v)
    lane = lax.iota(jnp.int32, 16)
    NCHUNK = B_PER_W // CH
    RQ = CH // NQ

    def issue_chunk(k):
        b = k % 2
        pltpu.async_copy(wpos_hbm.at[pl.ds(base + k * CH, CH)],
                         pos_b[b], psem_b[b])
        for q in range(NQ):

            @plsc.parallel_loop(0, RQ, unroll=4)
            def _fetch_row(r):
                rk = k * CH + q * RQ + r
                vec = idx_v[pl.ds((rk // 16) * 16, 16)]
                i = jnp.sum(jnp.where(lane == (rk % 16), vec, 0))
                pltpu.async_copy(wword_hbm.at[i],
                                 rows_b[b].at[q * RQ + r], gsems[b * NQ + q])

    def finish_chunk(k):
        b = k % 2
        if k >= 2:
            pltpu.make_async_copy(
                buf_b[b], out_hbm.at[pl.ds(0, CH)], wsem).wait()
        for q in range(NQ):
            pltpu.make_async_copy(
                wword_hbm.at[pl.ds(0, RQ)],
                rows_b[b].at[pl.ds(q * RQ, RQ)],
                gsems[b * NQ + q]).wait()
        pltpu.make_async_copy(
            wpos_hbm.at[pl.ds(0, CH)], pos_b[b], psem_b[b]).wait()

        @plsc.parallel_loop(0, CH, unroll=4)
        def _interleave(r):
            for c in range(WDIM // 16):
                buf_b[b][r, pl.ds(c * 16, 16)] = (
                    rows_b[b][r, pl.ds(c * 16, 16)])
            for c in range(PDIM // 16):
                buf_b[b][r, pl.ds(WDIM + c * 16, 16)] = (
                    pos_b[b][r, pl.ds(c * 16, 16)])

        pltpu.async_copy(buf_b[b], out_hbm.at[pl.ds(base + k * CH, CH)], wsem)

    issue_chunk(0)
    for k in range(1, NCHUNK):
        issue_chunk(k)
        finish_chunk(k - 1)
    finish_chunk(NCHUNK - 1)
    for _ in range(min(2, NCHUNK)):
        pltpu.make_async_copy(
            buf0_v, out_hbm.at[pl.ds(0, CH)], wsem).wait()


TC_ROWS = L_SEQ - SC_ROWS
NSEM = 16
TCH = 1024


def _tc_body(idx_sm, wword_hbm, wpos_hbm, out_hbm, buf_v, rows_v, pos_v,
             dsems, psem, wsem):
    NCHUNK = TC_ROWS // TCH

    def do_chunk(k, _):
        pltpu.make_async_copy(
            wpos_hbm.at[pl.ds(SC_ROWS + k * TCH, TCH)], pos_v, psem).start()

        def issue(r, _):
            i = idx_sm[k * TCH + r]
            pltpu.make_async_copy(
                wword_hbm.at[pl.ds(i, 1)],
                rows_v.at[pl.ds(r, 1)],
                dsems.at[lax.rem(r, NSEM)]).start()
            return 0

        lax.fori_loop(0, TCH, issue, 0)
        for q in range(NSEM):
            pltpu.make_async_copy(
                wword_hbm.at[pl.ds(0, TCH // NSEM)],
                rows_v.at[pl.ds(0, TCH // NSEM)],
                dsems.at[q]).wait()
        pltpu.make_async_copy(
            wpos_hbm.at[pl.ds(0, TCH)], pos_v, psem).wait()
        buf_v[:, pl.ds(0, WDIM)] = rows_v[...]
        buf_v[:, pl.ds(WDIM, PDIM)] = pos_v[...]
        pltpu.make_async_copy(
            buf_v, out_hbm.at[pl.ds(k * TCH, TCH)], wsem).start()
        pltpu.make_async_copy(
            buf_v, out_hbm.at[pl.ds(0, TCH)], wsem).wait()
        return 0

    lax.fori_loop(0, NCHUNK, do_chunk, 0, unroll=False)


_embed_tc = pl.pallas_call(
    _tc_body,
    out_shape=jax.ShapeDtypeStruct((TC_ROWS, ODIM), jnp.float32),
    in_specs=[
        pl.BlockSpec(memory_space=pltpu.SMEM),
        pl.BlockSpec(memory_space=pl.ANY),
        pl.BlockSpec(memory_space=pl.ANY),
    ],
    out_specs=pl.BlockSpec(memory_space=pl.ANY),
    scratch_shapes=[
        pltpu.VMEM((TCH, ODIM), jnp.float32),
        pltpu.VMEM((TCH, WDIM), jnp.float32),
        pltpu.VMEM((TCH, PDIM), jnp.float32),
        pltpu.SemaphoreType.DMA((NSEM,)),
        pltpu.SemaphoreType.DMA,
        pltpu.SemaphoreType.DMA,
    ],
)


def kernel(input, W_word, W_pos):
    idx = input.astype(jnp.int32)
    out_sc = _embed_sc(idx[:SC_ROWS], W_word, W_pos)
    out_tc = _embed_tc(idx[SC_ROWS:], W_word, W_pos)
    return jnp.concatenate([out_sc, out_tc], axis=0)


# speedup vs baseline: 1.0057x; 1.0057x over previous
import functools

import jax
import jax.numpy as jnp
from jax import lax
from jax.experimental import pallas as pl
from jax.experimental.pallas import tpu as pltpu

L_SEQ = 16384
WDIM = 64
PDIM = 64
ODIM = WDIM + PDIM
NSEM = 16
CH = 2048


def _tc_kernel(idx_sm, wword_hbm, wpos_hbm, out_hbm, buf_v, rows_v, pos_v,
               dsems, psem, wsem):
    NCHUNK = L_SEQ // CH

    def do_chunk(k, _):
        pltpu.make_async_copy(
            wpos_hbm.at[pl.ds(k * CH, CH)], pos_v, psem).start()

        def issue(g, _):
            for u in range(NSEM):
                r = g * NSEM + u
                i = idx_sm[k * CH + r]
                pltpu.make_async_copy(
                    wword_hbm.at[pl.ds(i, 1)],
                    rows_v.at[pl.ds(r, 1)],
                    dsems.at[u]).start()
            return 0

        lax.fori_loop(0, CH // NSEM, issue, 0)
        for q in range(NSEM):
            pltpu.make_async_copy(
                wword_hbm.at[pl.ds(0, CH // NSEM)],
                rows_v.at[pl.ds(0, CH // NSEM)],
                dsems.at[q]).wait()
        pltpu.make_async_copy(
            wpos_hbm.at[pl.ds(0, CH)], pos_v, psem).wait()
        buf_v[:, pl.ds(0, WDIM)] = rows_v[...]
        buf_v[:, pl.ds(WDIM, PDIM)] = pos_v[...]
        pltpu.make_async_copy(
            buf_v, out_hbm.at[pl.ds(k * CH, CH)], wsem).start()
        pltpu.make_async_copy(
            buf_v, out_hbm.at[pl.ds(0, CH)], wsem).wait()
        return 0

    lax.fori_loop(0, NCHUNK, do_chunk, 0, unroll=False)


def kernel(input, W_word, W_pos):
    idx = input.astype(jnp.int32)
    f = pl.pallas_call(
        _tc_kernel,
        out_shape=jax.ShapeDtypeStruct((L_SEQ, ODIM), jnp.float32),
        in_specs=[
            pl.BlockSpec(memory_space=pltpu.SMEM),
            pl.BlockSpec(memory_space=pl.ANY),
            pl.BlockSpec(memory_space=pl.ANY),
        ],
        out_specs=pl.BlockSpec(memory_space=pl.ANY),
        scratch_shapes=[
            pltpu.VMEM((CH, ODIM), jnp.float32),
            pltpu.VMEM((CH, WDIM), jnp.float32),
            pltpu.VMEM((CH, PDIM), jnp.float32),
            pltpu.SemaphoreType.DMA((NSEM,)),
            pltpu.SemaphoreType.DMA,
            pltpu.SemaphoreType.DMA,
        ],
    )
    return f(idx, W_word, W_pos)


# speedup vs baseline: 1.1518x; 1.1452x over previous
import functools

import jax
import jax.numpy as jnp
from jax import lax
from jax.experimental import pallas as pl
from jax.experimental.pallas import tpu as pltpu
from jax.experimental.pallas import tpu_sc as plsc

L_SEQ = 16384
WDIM = 64
PDIM = 64
ODIM = WDIM + PDIM

NUM_CORES = 2
NUM_SUBCORES = 16
NW = NUM_CORES * NUM_SUBCORES
B_PER_W = L_SEQ // NW
CH = 128
NQ = 4

_mesh = plsc.VectorSubcoreMesh(core_axis_name="c", subcore_axis_name="s")


@functools.partial(
    pl.kernel,
    mesh=_mesh,
    out_type=jax.ShapeDtypeStruct((L_SEQ, ODIM), jnp.float32),
    compiler_params=pltpu.CompilerParams(needs_layout_passes=False),
    scratch_types=[
        pltpu.VMEM((B_PER_W,), jnp.int32),
        pltpu.VMEM((CH, WDIM), jnp.float32),
        pltpu.VMEM((CH, WDIM), jnp.float32),
        pltpu.VMEM((CH, PDIM), jnp.float32),
        pltpu.VMEM((CH, PDIM), jnp.float32),
        pltpu.VMEM((CH, ODIM), jnp.float32),
        pltpu.VMEM((CH, ODIM), jnp.float32),
        [pltpu.SemaphoreType.DMA] * (2 * NQ),
        pltpu.SemaphoreType.DMA,
        pltpu.SemaphoreType.DMA,
        pltpu.SemaphoreType.DMA,
    ],
)
def _embed_sc(idx_hbm, wword_hbm, wpos_hbm, out_hbm,
              idx_v, rows0_v, rows1_v, pos0_v, pos1_v, buf0_v, buf1_v,
              gsems, psem0, psem1, wsem):
    rows_b = (rows0_v, rows1_v)
    pos_b = (pos0_v, pos1_v)
    buf_b = (buf0_v, buf1_v)
    psem_b = (psem0, psem1)
    wid = lax.axis_index("s") * NUM_CORES + lax.axis_index("c")
    base = wid * B_PER_W
    pltpu.sync_copy(idx_hbm.at[pl.ds(base, B_PER_W)], idx_v)
    lane = lax.iota(jnp.int32, 16)
    NCHUNK = B_PER_W // CH
    RQ = CH // NQ

    def issue_chunk(k):
        b = k % 2
        pltpu.async_copy(wpos_hbm.at[pl.ds(base + k * CH, CH)],
                         pos_b[b], psem_b[b])
        for q in range(NQ):

            @plsc.parallel_loop(0, RQ, unroll=4)
            def _fetch_row(r):
                rk = k * CH + q * RQ + r
                vec = idx_v[pl.ds((rk // 16) * 16, 16)]
                i = jnp.sum(jnp.where(lane == (rk % 16), vec, 0))
                pltpu.async_copy(wword_hbm.at[i],
                                 rows_b[b].at[q * RQ + r], gsems[b * NQ + q])

    def finish_chunk(k):
        b = k % 2
        if k >= 2:
            pltpu.make_async_copy(
                buf_b[b], out_hbm.at[pl.ds(0, CH)], wsem).wait()
        for q in range(NQ):
            pltpu.make_async_copy(
                wword_hbm.at[pl.ds(0, RQ)],
                rows_b[b].at[pl.ds(q * RQ, RQ)],
                gsems[b * NQ + q]).wait()
        pltpu.make_async_copy(
            wpos_hbm.at[pl.ds(0, CH)], pos_b[b], psem_b[b]).wait()

        @plsc.parallel_loop(0, CH, unroll=4)
        def _interleave(r):
            for c in range(WDIM // 16):
                buf_b[b][r, pl.ds(c * 16, 16)] = (
                    rows_b[b][r, pl.ds(c * 16, 16)])
            for c in range(PDIM // 16):
                buf_b[b][r, pl.ds(WDIM + c * 16, 16)] = (
                    pos_b[b][r, pl.ds(c * 16, 16)])

        pltpu.async_copy(buf_b[b], out_hbm.at[pl.ds(base + k * CH, CH)], wsem)

    issue_chunk(0)
    for k in range(1, NCHUNK):
        issue_chunk(k)
        finish_chunk(k - 1)
    finish_chunk(NCHUNK - 1)
    for _ in range(2):
        pltpu.make_async_copy(
            buf0_v, out_hbm.at[pl.ds(0, CH)], wsem).wait()


def kernel(input, W_word, W_pos):
    idx = input.astype(jnp.int32)
    return _embed_sc(idx, W_word, W_pos)
